# baseline (speedup 1.0000x reference)
import dataclasses
import functools

import jax
import jax.numpy as jnp
from jax import lax
from jax.experimental import pallas as pl
from jax.experimental.pallas import tpu as pltpu
from jax.experimental.pallas import tpu_sc as plsc

N = 10000
E = 320000
D = 128
OUT = 128

NC = 2
NS = 16
NW = NC * NS
CH = 128
CPW = 80
SB = 8
NCHUNKS = NW * CPW
E_PAD = NCHUNKS * CH
STEP = 2 * SB
ROWS_PER_SUB = 624
TAIL_ROWS = N - NS * ROWS_PER_SUB
LANES = 16
EPI = 8


def _sc_aggregate(x, src2, dst2, wt2, zeros):
    mesh = plsc.VectorSubcoreMesh(core_axis_name="c", subcore_axis_name="s")
    cp = pltpu.CompilerParams()
    if "needs_layout_passes" in pltpu.CompilerParams.__dataclass_fields__:
        cp = dataclasses.replace(cp, needs_layout_passes=False)

    @functools.partial(
        pl.kernel,
        out_type=jax.ShapeDtypeStruct((NC, N, D), jnp.float32),
        mesh=mesh,
        compiler_params=cp,
        scratch_types=[
            pltpu.VMEM((SB, CH), jnp.int32),
            pltpu.VMEM((SB, CH), jnp.int32),
            pltpu.VMEM((SB, CH), jnp.float32),
            pltpu.VMEM((SB, CH), jnp.int32),
            pltpu.VMEM((SB, CH), jnp.int32),
            pltpu.VMEM((SB, CH), jnp.float32),
            pltpu.VMEM((CH, D), jnp.float32),
            pltpu.VMEM((CH, D), jnp.float32),
            pltpu.VMEM_SHARED((N, D), jnp.float32),
        ] + [pltpu.SemaphoreType.DMA] * 6,
    )
    def agg(x_hbm, src_hbm, dst_hbm, wt_hbm, z_hbm, hp_hbm,
            s0, d0, w0, s1, d1, w1, r0, r1, h_sh,
            gsem0, gsem1, ssem0, ssem1, isem0, isem1):
        srcb = (s0, s1)
        dstb = (d0, d1)
        wtb = (w0, w1)
        rows = (r0, r1)
        gsems = (gsem0, gsem1)
        ssems = (ssem0, ssem1)
        isems = (isem0, isem1)

        cid = lax.axis_index("c")
        sid = lax.axis_index("s")
        wid = sid * NC + cid
        row0 = sid * ROWS_PER_SUB
        base = wid * CPW

        pltpu.sync_copy(z_hbm.at[pl.ds(row0, ROWS_PER_SUB)],
                        h_sh.at[pl.ds(row0, ROWS_PER_SUB)])

        @pl.when(sid == NS - 1)
        def _zero_tail():
            pltpu.sync_copy(z_hbm.at[pl.ds(NS * ROWS_PER_SUB, TAIL_ROWS)],
                            h_sh.at[pl.ds(NS * ROWS_PER_SUB, TAIL_ROWS)])

        plsc.subcore_barrier()

        def issue_idxset(first_chunk, p):
            sl = pl.ds(base + first_chunk, SB)
            pltpu.async_copy(src_hbm.at[sl], srcb[p], isems[p])
            pltpu.async_copy(dst_hbm.at[sl], dstb[p], isems[p])
            pltpu.async_copy(wt_hbm.at[sl], wtb[p], isems[p])

        def wait_idxset(p):
            sl = pl.ds(base, SB)
            pltpu.make_async_copy(src_hbm.at[sl], srcb[p], isems[p]).wait()
            pltpu.make_async_copy(dst_hbm.at[sl], dstb[p], isems[p]).wait()
            pltpu.make_async_copy(wt_hbm.at[sl], wtb[p], isems[p]).wait()

        def issue_gather(p, j, b):
            pltpu.async_copy(x_hbm.at[srcb[p].at[j]], rows[b], gsems[b])

        def wait_gather(p, j, b):
            pltpu.make_async_copy(x_hbm.at[srcb[p].at[j]], rows[b],
                                  gsems[b]).wait()

        def issue_scatter(p, j, b):
            pltpu.async_copy(rows[b], h_sh.at[dstb[p].at[j]], ssems[b],
                             add=True)

        def drain_scatter(p, j, b):
            pltpu.make_async_copy(rows[b], h_sh.at[dstb[p].at[j]],
                                  ssems[b]).wait()

        def scale_rows(p, j, b):
            buf = rows[b]
            wts = wtb[p]
            jv = jnp.full((LANES,), j, jnp.int32)

            @pl.loop(0, CH // EPI)
            def _it(it):
                for jj in range(EPI):
                    e = it * EPI + jj
                    w = plsc.load_gather(
                        wts, [jv, jnp.full((LANES,), e, jnp.int32)])
                    for dd in range(D // LANES):
                        sl = pl.ds(dd * LANES, LANES)
                        buf[e, sl] = buf[e, sl] * w

        issue_idxset(0, 0)
        wait_idxset(0)
        issue_gather(0, 0, 0)

        @pl.loop(0, CPW, step=STEP)
        def _body(c):
            for k in range(STEP):
                l = c + k
                b = k % 2
                p = k // SB
                j = k % SB

                @pl.when(l + 1 < CPW)
                def _prefetch():
                    if k % SB == SB - 1:
                        wait_idxset(1 - p)

                    @pl.when(l >= 1)
                    def _drain():
                        kp = k - 1
                        drain_scatter((kp // SB) % 2, kp % SB, 1 - b)

                    kn = k + 1
                    issue_gather((kn // SB) % 2, kn % SB, 1 - b)

                if k % SB == 0:
                    @pl.when(c + k + 2 * SB <= CPW)
                    def _idx_ahead():
                        issue_idxset(l + SB, 1 - p)

                wait_gather(p, j, b)
                scale_rows(p, j, b)
                issue_scatter(p, j, b)

        for k in range(CPW - 2, CPW):
            kk = k % STEP
            drain_scatter(kk // SB, kk % SB, k % 2)

        plsc.subcore_barrier()
        pltpu.sync_copy(h_sh.at[pl.ds(row0, ROWS_PER_SUB)],
                        hp_hbm.at[cid, pl.ds(row0, ROWS_PER_SUB)])

        @pl.when(sid == NS - 1)
        def _flush_tail():
            pltpu.sync_copy(h_sh.at[pl.ds(NS * ROWS_PER_SUB, TAIL_ROWS)],
                            hp_hbm.at[cid, pl.ds(NS * ROWS_PER_SUB, TAIL_ROWS)])

    return agg(x, src2, dst2, wt2, zeros)


def _tc_project(h0, h1, x, wt, wb):
    RB = 1000

    def body(h0_ref, h1_ref, x_ref, wt_ref, wb_ref, o_ref):
        h = h0_ref[...] + h1_ref[...]
        acc = jnp.dot(h, wt_ref[...], preferred_element_type=jnp.float32)
        acc = acc + jnp.dot(x_ref[...], wb_ref[...],
                            preferred_element_type=jnp.float32)
        o_ref[...] = jnp.maximum(acc, 0.0)

    return pl.pallas_call(
        body,
        grid=(N // RB,),
        in_specs=[
            pl.BlockSpec((RB, D), lambda i: (i, 0)),
            pl.BlockSpec((RB, D), lambda i: (i, 0)),
            pl.BlockSpec((RB, D), lambda i: (i, 0)),
            pl.BlockSpec((D, OUT), lambda i: (0, 0)),
            pl.BlockSpec((D, OUT), lambda i: (0, 0)),
        ],
        out_specs=pl.BlockSpec((RB, OUT), lambda i: (i, 0)),
        out_shape=jax.ShapeDtypeStruct((N, OUT), jnp.float32),
    )(h0, h1, x, wt, wb)


def kernel(x, edge_index, edge_weight, W):
    pad = E_PAD - E
    src2 = jnp.concatenate(
        [edge_index[1], jnp.zeros((pad,), jnp.int32)]).reshape(NCHUNKS, CH)
    dst2 = jnp.concatenate(
        [edge_index[0], jnp.zeros((pad,), jnp.int32)]).reshape(NCHUNKS, CH)
    wt2 = jnp.concatenate(
        [edge_weight, jnp.zeros((pad,), jnp.float32)]).reshape(NCHUNKS, CH)
    zeros = jnp.zeros((N, D), jnp.float32)
    hp = _sc_aggregate(x, src2, dst2, wt2, zeros)
    return _tc_project(hp[0], hp[1], x, W[:D], W[D:])
